# baseline (speedup 1.0000x reference)
import functools

import jax
import jax.numpy as jnp
from jax import lax
from jax.experimental import pallas as pl
from jax.experimental.pallas import tpu as pltpu
from jax.experimental.pallas import tpu_sc as plsc

HIDDEN = 768
EPS = 1e-12

_NUM_CORES = 2
_NUM_SUBCORES = 16
_NUM_WORKERS = _NUM_CORES * _NUM_SUBCORES
_CHUNK = 64


def _sc_gather(table, idx):
    num_idx = idx.shape[0]
    d = table.shape[1]
    b_per_w = num_idx // _NUM_WORKERS
    n_chunks = b_per_w // _CHUNK
    mesh = plsc.VectorSubcoreMesh(core_axis_name="c", subcore_axis_name="s")

    @functools.partial(
        pl.kernel,
        mesh=mesh,
        out_type=jax.ShapeDtypeStruct((num_idx, d), table.dtype),
        scratch_types=[
            pltpu.VMEM((b_per_w,), jnp.int32),
            pltpu.VMEM((_CHUNK, d), table.dtype),
            pltpu.VMEM((_CHUNK, d), table.dtype),
            pltpu.SemaphoreType.DMA,
            pltpu.SemaphoreType.DMA,
            pltpu.SemaphoreType.DMA,
            pltpu.SemaphoreType.DMA,
        ],
    )
    def gather_kernel(table_hbm, idx_hbm, out_hbm, idx_v, rows0,
                      rows1, gsem0, gsem1, wsem0, wsem1):
        wid = lax.axis_index("s") * _NUM_CORES + lax.axis_index("c")
        base = wid * b_per_w
        rows = (rows0, rows1)
        gsems = (gsem0, gsem1)
        wsems = (wsem0, wsem1)

        pltpu.sync_copy(idx_hbm.at[pl.ds(base, b_per_w)], idx_v)

        gh = {}
        wh = {}

        def start_gather(i):
            b = i & 1
            gh[i] = pltpu.async_copy(
                table_hbm.at[idx_v.at[pl.ds(i * _CHUNK, _CHUNK)]], rows[b],
                gsems[b])

        start_gather(0)
        for i in range(n_chunks):
            b = i & 1
            if i + 1 < n_chunks:
                if i - 1 >= 0:
                    wh[i - 1].wait()
                start_gather(i + 1)
            gh[i].wait()
            wh[i] = pltpu.async_copy(
                rows[b], out_hbm.at[pl.ds(base + i * _CHUNK, _CHUNK)],
                wsems[b])
        if n_chunks >= 2:
            wh[n_chunks - 2].wait()
        wh[n_chunks - 1].wait()

    return gather_kernel(table, idx)


def _fused_body(g_ref, pos_ref, type_ref, gamma_ref, beta_ref, wt_ref, b_ref,
                out_ref):
    x = g_ref[...] + pos_ref[...] + type_ref[...]
    mean = jnp.mean(x, axis=1, keepdims=True)
    xc = x - mean
    var = jnp.mean(xc * xc, axis=1, keepdims=True)
    y = xc * lax.rsqrt(var + EPS) * gamma_ref[...] + beta_ref[...]
    out_ref[...] = lax.dot_general(
        y.astype(jnp.bfloat16), wt_ref[...], (((1,), (0,)), ((), ())),
        preferred_element_type=jnp.float32) + b_ref[...]


def _tc_fused(gathered, pos_emb, type_emb, ln_gamma, ln_beta, proj_Wt, proj_b,
              token_block):
    n_tok, d = gathered.shape
    seq = pos_emb.shape[0]
    batch = n_tok // seq
    pos_blocks = seq // token_block
    return pl.pallas_call(
        _fused_body,
        grid=(pos_blocks, batch),
        in_specs=[
            pl.BlockSpec((token_block, d), lambda p, b: (b * pos_blocks + p, 0)),
            pl.BlockSpec((token_block, d), lambda p, b: (p, 0)),
            pl.BlockSpec((1, d), lambda p, b: (0, 0)),
            pl.BlockSpec((1, d), lambda p, b: (0, 0)),
            pl.BlockSpec((1, d), lambda p, b: (0, 0)),
            pl.BlockSpec((d, d), lambda p, b: (0, 0)),
            pl.BlockSpec((1, d), lambda p, b: (0, 0)),
        ],
        out_specs=pl.BlockSpec((token_block, d),
                               lambda p, b: (b * pos_blocks + p, 0)),
        out_shape=jax.ShapeDtypeStruct((n_tok, d), jnp.float32),
    )(gathered, pos_emb, type_emb, ln_gamma, ln_beta, proj_Wt, proj_b)


def kernel(input_ids, word_emb, pos_emb, type_emb, ln_gamma, ln_beta, proj_W,
           proj_b):
    batch, seq = input_ids.shape
    ids = input_ids.reshape(-1).astype(jnp.int32)
    gamma = ln_gamma.reshape(1, -1)
    beta = ln_beta.reshape(1, -1)
    bias = proj_b.reshape(1, -1)
    type0 = type_emb[:1, :]
    wt = proj_W.T.astype(jnp.bfloat16)
    half = batch * seq // 2
    g0 = _sc_gather(word_emb, lax.slice(ids, (0,), (half,)))
    g1 = _sc_gather(word_emb, lax.slice(ids, (half,), (2 * half,)))
    t0 = _tc_fused(g0, pos_emb, type0, gamma, beta, wt, bias,
                   token_block=1024)
    t1 = _tc_fused(g1, pos_emb, type0, gamma, beta, wt, bias,
                   token_block=1024)
    return jnp.concatenate([t0, t1], axis=0).reshape(batch, seq, HIDDEN)


# speedup vs baseline: 1.3693x; 1.3693x over previous
import functools

import jax
import jax.numpy as jnp
from jax import lax
from jax.experimental import pallas as pl
from jax.experimental.pallas import tpu as pltpu
from jax.experimental.pallas import tpu_sc as plsc

HIDDEN = 768
EPS = 1e-12

_NUM_CORES = 2
_NUM_SUBCORES = 16
_NUM_WORKERS = _NUM_CORES * _NUM_SUBCORES
_CHUNK = 64


def _sc_gather(table, idx):
    num_idx = idx.shape[0]
    d = table.shape[1]
    b_per_w = num_idx // _NUM_WORKERS
    n_chunks = b_per_w // _CHUNK
    mesh = plsc.VectorSubcoreMesh(core_axis_name="c", subcore_axis_name="s")

    @functools.partial(
        pl.kernel,
        mesh=mesh,
        out_type=jax.ShapeDtypeStruct((num_idx, d), table.dtype),
        scratch_types=[
            pltpu.VMEM((b_per_w,), jnp.int32),
            pltpu.VMEM((_CHUNK, d), table.dtype),
            pltpu.VMEM((_CHUNK, d), table.dtype),
            pltpu.SemaphoreType.DMA,
            pltpu.SemaphoreType.DMA,
            pltpu.SemaphoreType.DMA,
            pltpu.SemaphoreType.DMA,
        ],
    )
    def gather_kernel(table_hbm, idx_hbm, out_hbm, idx_v, rows0,
                      rows1, gsem0, gsem1, wsem0, wsem1):
        wid = lax.axis_index("s") * _NUM_CORES + lax.axis_index("c")
        base = wid * b_per_w
        rows = (rows0, rows1)
        gsems = (gsem0, gsem1)
        wsems = (wsem0, wsem1)

        pltpu.sync_copy(idx_hbm.at[pl.ds(base, b_per_w)], idx_v)

        gh = {}
        wh = {}

        def start_gather(i):
            b = i & 1
            gh[i] = pltpu.async_copy(
                table_hbm.at[idx_v.at[pl.ds(i * _CHUNK, _CHUNK)]], rows[b],
                gsems[b])

        start_gather(0)
        for i in range(n_chunks):
            b = i & 1
            if i + 1 < n_chunks:
                if i - 1 >= 0:
                    wh[i - 1].wait()
                start_gather(i + 1)
            gh[i].wait()
            wh[i] = pltpu.async_copy(
                rows[b], out_hbm.at[pl.ds(base + i * _CHUNK, _CHUNK)],
                wsems[b])
        if n_chunks >= 2:
            wh[n_chunks - 2].wait()
        wh[n_chunks - 1].wait()

    return gather_kernel(table, idx)


def _fused_body(g_ref, pos_ref, type_ref, gamma_ref, beta_ref, wt_ref, b_ref,
                out_ref):
    x = g_ref[...] + pos_ref[...] + type_ref[...]
    mean = jnp.mean(x, axis=1, keepdims=True)
    xc = x - mean
    var = jnp.mean(xc * xc, axis=1, keepdims=True)
    y = xc * lax.rsqrt(var + EPS) * gamma_ref[...] + beta_ref[...]
    out_ref[...] = lax.dot_general(
        y.astype(jnp.bfloat16), wt_ref[...], (((1,), (0,)), ((), ())),
        preferred_element_type=jnp.float32) + b_ref[...]


def _tc_fused(gathered, pos_emb, type_emb, ln_gamma, ln_beta, proj_Wt, proj_b,
              token_block):
    n_tok, d = gathered.shape
    seq = pos_emb.shape[0]
    batch = n_tok // seq
    pos_blocks = seq // token_block
    return pl.pallas_call(
        _fused_body,
        grid=(pos_blocks, batch),
        in_specs=[
            pl.BlockSpec((token_block, d), lambda p, b: (b * pos_blocks + p, 0)),
            pl.BlockSpec((token_block, d), lambda p, b: (p, 0)),
            pl.BlockSpec((1, d), lambda p, b: (0, 0)),
            pl.BlockSpec((1, d), lambda p, b: (0, 0)),
            pl.BlockSpec((1, d), lambda p, b: (0, 0)),
            pl.BlockSpec((d, d), lambda p, b: (0, 0)),
            pl.BlockSpec((1, d), lambda p, b: (0, 0)),
        ],
        out_specs=pl.BlockSpec((token_block, d),
                               lambda p, b: (b * pos_blocks + p, 0)),
        out_shape=jax.ShapeDtypeStruct((n_tok, d), jnp.float32),
    )(gathered, pos_emb, type_emb, ln_gamma, ln_beta, proj_Wt, proj_b)


def kernel(input_ids, word_emb, pos_emb, type_emb, ln_gamma, ln_beta, proj_W,
           proj_b):
    batch, seq = input_ids.shape
    ids = input_ids.reshape(-1).astype(jnp.int32)
    gamma = ln_gamma.reshape(1, -1)
    beta = ln_beta.reshape(1, -1)
    bias = proj_b.reshape(1, -1)
    type0 = type_emb[:1, :]
    wt = proj_W.T.astype(jnp.bfloat16)
    gathered = _sc_gather(word_emb, ids)
    out = _tc_fused(gathered, pos_emb, type0, gamma, beta, wt, bias,
                    token_block=2048)
    return out.reshape(batch, seq, HIDDEN)


# speedup vs baseline: 1.3780x; 1.0063x over previous
import functools

import jax
import jax.numpy as jnp
from jax import lax
from jax.experimental import pallas as pl
from jax.experimental.pallas import tpu as pltpu
from jax.experimental.pallas import tpu_sc as plsc

HIDDEN = 768
EPS = 1e-12

_NUM_CORES = 2
_NUM_SUBCORES = 16
_NUM_WORKERS = _NUM_CORES * _NUM_SUBCORES
_CHUNK = 32
_NBUF = 4


def _sc_gather(table, idx):
    num_idx = idx.shape[0]
    d = table.shape[1]
    b_per_w = num_idx // _NUM_WORKERS
    n_chunks = b_per_w // _CHUNK
    mesh = plsc.VectorSubcoreMesh(core_axis_name="c", subcore_axis_name="s")

    @functools.partial(
        pl.kernel,
        mesh=mesh,
        out_type=jax.ShapeDtypeStruct((num_idx, d), table.dtype),
        scratch_types=(
            [pltpu.VMEM((b_per_w,), jnp.int32)]
            + [pltpu.VMEM((_CHUNK, d), table.dtype) for _ in range(_NBUF)]
            + [pltpu.SemaphoreType.DMA for _ in range(2 * _NBUF)]
        ),
    )
    def gather_kernel(table_hbm, idx_hbm, out_hbm, idx_v, *bufs):
        rows = bufs[:_NBUF]
        gsems = bufs[_NBUF:2 * _NBUF]
        wsems = bufs[2 * _NBUF:]
        wid = lax.axis_index("s") * _NUM_CORES + lax.axis_index("c")
        base = wid * b_per_w

        pltpu.sync_copy(idx_hbm.at[pl.ds(base, b_per_w)], idx_v)

        gh = {}
        wh = {}

        def start_gather(i):
            b = i % _NBUF
            gh[i] = pltpu.async_copy(
                table_hbm.at[idx_v.at[pl.ds(i * _CHUNK, _CHUNK)]], rows[b],
                gsems[b])

        for i in range(min(_NBUF, n_chunks)):
            start_gather(i)
        for i in range(n_chunks):
            b = i % _NBUF
            gh[i].wait()
            wh[i] = pltpu.async_copy(
                rows[b], out_hbm.at[pl.ds(base + i * _CHUNK, _CHUNK)],
                wsems[b])
            if i + _NBUF < n_chunks:
                wh[i].wait()
                start_gather(i + _NBUF)
        for i in range(max(0, n_chunks - _NBUF), n_chunks):
            wh[i].wait()

    return gather_kernel(table, idx)


def _fused_body(g_ref, pos_ref, type_ref, gamma_ref, beta_ref, wt_ref, b_ref,
                out_ref):
    x = g_ref[...] + pos_ref[...] + type_ref[...]
    mean = jnp.mean(x, axis=1, keepdims=True)
    xc = x - mean
    var = jnp.mean(xc * xc, axis=1, keepdims=True)
    y = xc * lax.rsqrt(var + EPS) * gamma_ref[...] + beta_ref[...]
    out_ref[...] = lax.dot_general(
        y.astype(jnp.bfloat16), wt_ref[...], (((1,), (0,)), ((), ())),
        preferred_element_type=jnp.float32) + b_ref[...]


def _tc_fused(gathered, pos_emb, type_emb, ln_gamma, ln_beta, proj_Wt, proj_b,
              token_block):
    n_tok, d = gathered.shape
    seq = pos_emb.shape[0]
    batch = n_tok // seq
    pos_blocks = seq // token_block
    return pl.pallas_call(
        _fused_body,
        grid=(pos_blocks, batch),
        in_specs=[
            pl.BlockSpec((token_block, d), lambda p, b: (b * pos_blocks + p, 0)),
            pl.BlockSpec((token_block, d), lambda p, b: (p, 0)),
            pl.BlockSpec((1, d), lambda p, b: (0, 0)),
            pl.BlockSpec((1, d), lambda p, b: (0, 0)),
            pl.BlockSpec((1, d), lambda p, b: (0, 0)),
            pl.BlockSpec((d, d), lambda p, b: (0, 0)),
            pl.BlockSpec((1, d), lambda p, b: (0, 0)),
        ],
        out_specs=pl.BlockSpec((token_block, d),
                               lambda p, b: (b * pos_blocks + p, 0)),
        out_shape=jax.ShapeDtypeStruct((n_tok, d), jnp.float32),
    )(gathered, pos_emb, type_emb, ln_gamma, ln_beta, proj_Wt, proj_b)


def kernel(input_ids, word_emb, pos_emb, type_emb, ln_gamma, ln_beta, proj_W,
           proj_b):
    batch, seq = input_ids.shape
    ids = input_ids.reshape(-1).astype(jnp.int32)
    gamma = ln_gamma.reshape(1, -1)
    beta = ln_beta.reshape(1, -1)
    bias = proj_b.reshape(1, -1)
    type0 = type_emb[:1, :]
    wt = proj_W.T.astype(jnp.bfloat16)
    gathered = _sc_gather(word_emb, ids)
    out = _tc_fused(gathered, pos_emb, type0, gamma, beta, wt, bias,
                    token_block=2048)
    return out.reshape(batch, seq, HIDDEN)


# speedup vs baseline: 1.3795x; 1.0011x over previous
import functools

import jax
import jax.numpy as jnp
from jax import lax
from jax.experimental import pallas as pl
from jax.experimental.pallas import tpu as pltpu
from jax.experimental.pallas import tpu_sc as plsc

HIDDEN = 768
EPS = 1e-12

_NUM_CORES = 2
_NUM_SUBCORES = 16
_NUM_WORKERS = _NUM_CORES * _NUM_SUBCORES
_CHUNK = 32
_NBUF = 4


def _sc_gather(table, idx):
    num_idx = idx.shape[0]
    d = table.shape[1]
    b_per_w = num_idx // _NUM_WORKERS
    n_chunks = b_per_w // _CHUNK
    mesh = plsc.VectorSubcoreMesh(core_axis_name="c", subcore_axis_name="s")

    @functools.partial(
        pl.kernel,
        mesh=mesh,
        out_type=jax.ShapeDtypeStruct((num_idx, d), table.dtype),
        scratch_types=(
            [pltpu.VMEM((b_per_w,), jnp.int32)]
            + [pltpu.VMEM((_CHUNK, d), table.dtype) for _ in range(_NBUF)]
            + [pltpu.SemaphoreType.DMA for _ in range(2 * _NBUF)]
        ),
    )
    def gather_kernel(table_hbm, idx_hbm, out_hbm, idx_v, *bufs):
        rows = bufs[:_NBUF]
        gsems = bufs[_NBUF:2 * _NBUF]
        wsems = bufs[2 * _NBUF:]
        wid = lax.axis_index("s") * _NUM_CORES + lax.axis_index("c")
        base = wid * b_per_w

        pltpu.sync_copy(idx_hbm.at[pl.ds(base, b_per_w)], idx_v)

        gh = {}
        wh = {}

        def start_gather(i):
            b = i % _NBUF
            gh[i] = pltpu.async_copy(
                table_hbm.at[idx_v.at[pl.ds(i * _CHUNK, _CHUNK)]], rows[b],
                gsems[b])

        for i in range(min(_NBUF, n_chunks)):
            start_gather(i)
        for i in range(n_chunks):
            b = i % _NBUF
            gh[i].wait()
            wh[i] = pltpu.async_copy(
                rows[b], out_hbm.at[pl.ds(base + i * _CHUNK, _CHUNK)],
                wsems[b])
            if i + _NBUF < n_chunks:
                wh[i].wait()
                start_gather(i + _NBUF)
        for i in range(max(0, n_chunks - _NBUF), n_chunks):
            wh[i].wait()

    return gather_kernel(table, idx)


def _fused_body(g_ref, pos_ref, type_ref, gamma_ref, beta_ref, wt_ref, b_ref,
                out_ref):
    x = g_ref[...] + pos_ref[...] + type_ref[...]
    inv_d = 1.0 / x.shape[1]
    mean = jnp.sum(x, axis=1, keepdims=True) * inv_d
    var = jnp.sum(x * x, axis=1, keepdims=True) * inv_d - mean * mean
    rstd = lax.rsqrt(var + EPS)
    y = (x - mean) * (rstd * gamma_ref[...]) + beta_ref[...]
    out_ref[...] = lax.dot_general(
        y.astype(jnp.bfloat16), wt_ref[...], (((1,), (0,)), ((), ())),
        preferred_element_type=jnp.float32) + b_ref[...]


def _tc_fused(gathered, pos_emb, type_emb, ln_gamma, ln_beta, proj_Wt, proj_b,
              token_block):
    n_tok, d = gathered.shape
    seq = pos_emb.shape[0]
    batch = n_tok // seq
    pos_blocks = seq // token_block
    return pl.pallas_call(
        _fused_body,
        grid=(pos_blocks, batch),
        in_specs=[
            pl.BlockSpec((token_block, d), lambda p, b: (b * pos_blocks + p, 0)),
            pl.BlockSpec((token_block, d), lambda p, b: (p, 0)),
            pl.BlockSpec((1, d), lambda p, b: (0, 0)),
            pl.BlockSpec((1, d), lambda p, b: (0, 0)),
            pl.BlockSpec((1, d), lambda p, b: (0, 0)),
            pl.BlockSpec((d, d), lambda p, b: (0, 0)),
            pl.BlockSpec((1, d), lambda p, b: (0, 0)),
        ],
        out_specs=pl.BlockSpec((token_block, d),
                               lambda p, b: (b * pos_blocks + p, 0)),
        out_shape=jax.ShapeDtypeStruct((n_tok, d), jnp.float32),
    )(gathered, pos_emb, type_emb, ln_gamma, ln_beta, proj_Wt, proj_b)


def kernel(input_ids, word_emb, pos_emb, type_emb, ln_gamma, ln_beta, proj_W,
           proj_b):
    batch, seq = input_ids.shape
    ids = input_ids.reshape(-1).astype(jnp.int32)
    gamma = ln_gamma.reshape(1, -1)
    beta = ln_beta.reshape(1, -1)
    bias = proj_b.reshape(1, -1)
    type0 = type_emb[:1, :]
    wt = proj_W.T.astype(jnp.bfloat16)
    gathered = _sc_gather(word_emb, ids)
    out = _tc_fused(gathered, pos_emb, type0, gamma, beta, wt, bias,
                    token_block=2048)
    return out.reshape(batch, seq, HIDDEN)
